# Initial kernel scaffold; baseline (speedup 1.0000x reference)
import jax
import jax.numpy as jnp
from jax.experimental import pallas as pl


def kernel(basic_block, edge_index, W1, b1, W2, b2):
    raise NotImplementedError("write your pallas kernel here")


# speedup vs baseline: 17.9971x; 17.9971x over previous
import functools

import jax
import jax.numpy as jnp
from jax import lax
from jax.experimental import pallas as pl
from jax.experimental.pallas import tpu as pltpu
from jax.experimental.pallas import tpu_sc as plsc

N = 10000
E = 320000
D = 128

NC = 2
NS = 16
NW = NC * NS
EPW = E // NW
K = 80
NCHUNK = EPW // K

NPAD = 10240
RPT = NPAD // NS
RSTAGE = 80
CB = 25
NBLK = NCHUNK // CB

_MESH = plsc.VectorSubcoreMesh(core_axis_name="c", subcore_axis_name="s")


def _zero_f32(ref, nrow, ncol):
    z = jnp.zeros((16,), jnp.float32)

    def body(i, _):
        for j in range(ncol // 16):
            ref[i, pl.ds(j * 16, 16)] = z
        return 0

    lax.fori_loop(0, nrow, body, 0, unroll=False)


@functools.partial(
    pl.kernel,
    out_type=jax.ShapeDtypeStruct((NC, NPAD, D), jnp.float32),
    mesh=_MESH,
    scratch_types=[
        pltpu.VMEM((NCHUNK, K), jnp.int32),
        pltpu.VMEM((K, D), jnp.float32),
        pltpu.VMEM((RSTAGE, D), jnp.float32),
        pltpu.VMEM_SHARED((NPAD, D), jnp.float32),
    ],
)
def _sc_count(dst_hbm, out_hbm, idx_v, ones_v, stage_v, acc_sh):
    c = lax.axis_index("c")
    s = lax.axis_index("s")
    wid = c * NS + s

    pltpu.sync_copy(dst_hbm.at[wid], idx_v)

    one = jnp.ones((16,), jnp.float32)

    def fill_ones(i, _):
        for j in range(D // 16):
            ones_v[i, pl.ds(j * 16, 16)] = one
        return 0

    lax.fori_loop(0, K, fill_ones, 0, unroll=False)

    _zero_f32(stage_v, RSTAGE, D)
    for j in range(RPT // RSTAGE):
        pltpu.sync_copy(stage_v, acc_sh.at[pl.ds(s * RPT + j * RSTAGE, RSTAGE)])
    plsc.subcore_barrier()

    def body(i, _):
        pltpu.sync_copy(ones_v, acc_sh.at[idx_v.at[i]], add=True)
        return 0

    lax.fori_loop(0, NCHUNK, body, 0, unroll=False)
    plsc.subcore_barrier()

    for j in range(RPT // RSTAGE):
        rs = pl.ds(s * RPT + j * RSTAGE, RSTAGE)
        pltpu.sync_copy(acc_sh.at[rs], stage_v)
        pltpu.sync_copy(stage_v, out_hbm.at[c, rs])


@functools.partial(
    pl.kernel,
    out_type=jax.ShapeDtypeStruct((NC, NPAD, D), jnp.float32),
    mesh=_MESH,
    scratch_types=[
        pltpu.VMEM((CB * K,), jnp.int32),
        pltpu.VMEM((NCHUNK, K), jnp.int32),
        pltpu.VMEM((K, D), jnp.float32),
        pltpu.VMEM((RSTAGE, D), jnp.float32),
        pltpu.VMEM_SHARED((NPAD, D), jnp.float32),
        pltpu.SemaphoreType.DMA,
    ],
)
def _sc_scatter(g_hbm, src_hbm, dst_hbm, out_hbm,
                src_v, dst_v, rows_v, stage_v, acc_sh, sem):
    c = lax.axis_index("c")
    s = lax.axis_index("s")
    wid = c * NS + s

    pltpu.sync_copy(dst_hbm.at[wid], dst_v)

    _zero_f32(stage_v, RSTAGE, D)
    for j in range(RPT // RSTAGE):
        pltpu.sync_copy(stage_v, acc_sh.at[pl.ds(s * RPT + j * RSTAGE, RSTAGE)])
    plsc.subcore_barrier()

    def blk_body(blk, _):
        pltpu.sync_copy(src_hbm.at[pl.ds(wid * EPW + blk * (CB * K), CB * K)],
                        src_v)

        def body(i, _):
            pltpu.async_copy(g_hbm.at[src_v.at[pl.ds(i * K, K)]],
                             rows_v, sem).wait()
            pltpu.sync_copy(rows_v, acc_sh.at[dst_v.at[blk * CB + i]],
                            add=True)
            return 0

        lax.fori_loop(0, CB, body, 0, unroll=False)
        return 0

    lax.fori_loop(0, NBLK, blk_body, 0, unroll=False)
    plsc.subcore_barrier()

    for j in range(RPT // RSTAGE):
        rs = pl.ds(s * RPT + j * RSTAGE, RSTAGE)
        pltpu.sync_copy(acc_sh.at[rs], stage_v)
        pltpu.sync_copy(stage_v, out_hbm.at[c, rs])


R = 1000


def _dinv_block(degp_ref):
    deg = degp_ref[0, :, 0:1] + degp_ref[1, :, 0:1] + 1.0
    return lax.rsqrt(deg)


def _tc1_body(x_ref, w_ref, degp_ref, g_ref):
    dinv = _dinv_block(degp_ref)
    g_ref[...] = jnp.dot(x_ref[...], w_ref[...],
                         preferred_element_type=jnp.float32) * dinv


def _tc2_body(ap_ref, g_ref, degp_ref, b_ref, w_ref, g2_ref):
    dinv = _dinv_block(degp_ref)
    pre = dinv * (ap_ref[0] + ap_ref[1] + g_ref[...]) + b_ref[...]
    x2 = jnp.maximum(pre, 0.0)
    g2_ref[...] = jnp.dot(x2, w_ref[...],
                          preferred_element_type=jnp.float32) * dinv


def _tc3_body(ap_ref, g_ref, degp_ref, b_ref, out_ref):
    dinv = _dinv_block(degp_ref)
    out_ref[...] = dinv * (ap_ref[0] + ap_ref[1] + g_ref[...]) + b_ref[...]


def _row_spec(r):
    return pl.BlockSpec((r, D), lambda i: (i, 0))


_pair_spec = pl.BlockSpec((NC, R, D), lambda i: (0, i, 0))
_degp_spec = _pair_spec
_full_w = pl.BlockSpec((D, D), lambda i: (0, 0))
_full_b = pl.BlockSpec((1, D), lambda i: (0, 0))
_out_rd = jax.ShapeDtypeStruct((N, D), jnp.float32)

_tc1 = pl.pallas_call(
    _tc1_body,
    grid=(N // R,),
    in_specs=[_row_spec(R), _full_w, _degp_spec],
    out_specs=_row_spec(R),
    out_shape=_out_rd,
)

_tc2 = pl.pallas_call(
    _tc2_body,
    grid=(N // R,),
    in_specs=[_pair_spec, _row_spec(R), _degp_spec, _full_b, _full_w],
    out_specs=_row_spec(R),
    out_shape=_out_rd,
)

_tc3 = pl.pallas_call(
    _tc3_body,
    grid=(N // R,),
    in_specs=[_pair_spec, _row_spec(R), _degp_spec, _full_b],
    out_specs=_row_spec(R),
    out_shape=_out_rd,
)


def kernel(basic_block, edge_index, W1, b1, W2, b2):
    src1 = edge_index[0]
    dst3 = edge_index[1].reshape(NW, NCHUNK, K)
    b1r = b1.reshape(1, D)
    b2r = b2.reshape(1, D)

    degp = _sc_count(dst3)
    g1 = _tc1(basic_block, W1, degp)
    a1p = _sc_scatter(g1, src1, dst3)
    g2 = _tc2(a1p, g1, degp, b1r, W2)
    a2p = _sc_scatter(g2, src1, dst3)
    return _tc3(a2p, g2, degp, b2r)


# speedup vs baseline: 21.3470x; 1.1861x over previous
import functools

import jax
import jax.numpy as jnp
from jax import lax
from jax.experimental import pallas as pl
from jax.experimental.pallas import tpu as pltpu
from jax.experimental.pallas import tpu_sc as plsc

N = 10000
E = 320000
D = 128

NC = 2
NS = 16
NW = NC * NS
EPW = E // NW
K = 80
NCHUNK = EPW // K

NPAD = 10240
RPT = NPAD // NS
RSTAGE = 40
CB = 25
NBLK = NCHUNK // CB
NPAIR = (CB - 1) // 2

_MESH = plsc.VectorSubcoreMesh(core_axis_name="c", subcore_axis_name="s")


def _zero_f32(ref, nrow, ncol):
    z = jnp.zeros((16,), jnp.float32)

    def body(i, _):
        for j in range(ncol // 16):
            ref[i, pl.ds(j * 16, 16)] = z
        return 0

    lax.fori_loop(0, nrow, body, 0, unroll=False)


@functools.partial(
    pl.kernel,
    out_type=jax.ShapeDtypeStruct((NC, NPAD, D), jnp.float32),
    mesh=_MESH,
    scratch_types=[
        pltpu.VMEM((NCHUNK, K), jnp.int32),
        pltpu.VMEM((K, D), jnp.float32),
        pltpu.VMEM((RSTAGE, D), jnp.float32),
        pltpu.VMEM_SHARED((NPAD, D), jnp.float32),
    ],
)
def _sc_count(dst_hbm, out_hbm, idx_v, ones_v, stage_v, acc_sh):
    c = lax.axis_index("c")
    s = lax.axis_index("s")
    wid = c * NS + s

    pltpu.sync_copy(dst_hbm.at[wid], idx_v)

    one = jnp.ones((16,), jnp.float32)

    def fill_ones(i, _):
        for j in range(D // 16):
            ones_v[i, pl.ds(j * 16, 16)] = one
        return 0

    lax.fori_loop(0, K, fill_ones, 0, unroll=False)

    _zero_f32(stage_v, RSTAGE, D)
    for j in range(RPT // RSTAGE):
        pltpu.sync_copy(stage_v, acc_sh.at[pl.ds(s * RPT + j * RSTAGE, RSTAGE)])
    plsc.subcore_barrier()

    def body(i, _):
        pltpu.sync_copy(ones_v, acc_sh.at[idx_v.at[i]], add=True)
        return 0

    lax.fori_loop(0, NCHUNK, body, 0, unroll=False)
    plsc.subcore_barrier()

    for j in range(RPT // RSTAGE):
        rs = pl.ds(s * RPT + j * RSTAGE, RSTAGE)
        pltpu.sync_copy(acc_sh.at[rs], stage_v)
        pltpu.sync_copy(stage_v, out_hbm.at[c, rs])


@functools.partial(
    pl.kernel,
    out_type=jax.ShapeDtypeStruct((NC, NPAD, D), jnp.float32),
    mesh=_MESH,
    scratch_types=[
        pltpu.VMEM((CB, K), jnp.int32),
        pltpu.VMEM((CB, K), jnp.int32),
        pltpu.VMEM((K, D), jnp.float32),
        pltpu.VMEM((K, D), jnp.float32),
        pltpu.VMEM((RSTAGE, D), jnp.float32),
        pltpu.VMEM_SHARED((NPAD, D), jnp.float32),
        pltpu.SemaphoreType.DMA,
        pltpu.SemaphoreType.DMA,
    ],
)
def _sc_scatter(g_hbm, src_hbm, dst_hbm, out_hbm,
                src_v, dst_v, rows0, rows1, stage_v, acc_sh, gsem, ssem):
    c = lax.axis_index("c")
    s = lax.axis_index("s")
    wid = c * NS + s

    _zero_f32(stage_v, RSTAGE, D)
    for j in range(RPT // RSTAGE):
        pltpu.sync_copy(stage_v, acc_sh.at[pl.ds(s * RPT + j * RSTAGE, RSTAGE)])
    plsc.subcore_barrier()

    def g_start(i, buf):
        pltpu.make_async_copy(g_hbm.at[src_v.at[i]], buf, gsem).start()

    def g_wait(i, buf):
        pltpu.make_async_copy(g_hbm.at[src_v.at[i]], buf, gsem).wait()

    def s_start(i, buf):
        pltpu.make_async_copy(buf, acc_sh.at[dst_v.at[i]], ssem).start(add=True)

    def s_wait(i, buf):
        pltpu.make_async_copy(buf, acc_sh.at[dst_v.at[i]], ssem).wait()

    for blk in range(NBLK):
        pltpu.sync_copy(src_hbm.at[wid, blk], src_v)
        pltpu.sync_copy(dst_hbm.at[wid, blk], dst_v)

        g_start(0, rows0)
        g_wait(0, rows0)
        g_start(1, rows1)
        s_start(0, rows0)

        def pair(p, _):
            i1 = 2 * p + 1
            g_wait(i1, rows1)
            s_wait(i1 - 1, rows0)
            g_start(i1 + 1, rows0)
            s_start(i1, rows1)
            i2 = 2 * p + 2
            g_wait(i2, rows0)
            s_wait(i1, rows1)

            @pl.when(p < NPAIR - 1)
            def _():
                g_start(i2 + 1, rows1)

            s_start(i2, rows0)
            return 0

        lax.fori_loop(0, NPAIR, pair, 0, unroll=False)
        s_wait(CB - 1, rows0)
    plsc.subcore_barrier()

    for j in range(RPT // RSTAGE):
        rs = pl.ds(s * RPT + j * RSTAGE, RSTAGE)
        pltpu.sync_copy(acc_sh.at[rs], stage_v)
        pltpu.sync_copy(stage_v, out_hbm.at[c, rs])


R = 1000


def _dinv_block(degp_ref):
    deg = degp_ref[0, :, 0:1] + degp_ref[1, :, 0:1] + 1.0
    return lax.rsqrt(deg)


def _tc1_body(x_ref, w_ref, degp_ref, g_ref):
    dinv = _dinv_block(degp_ref)
    g_ref[...] = jnp.dot(x_ref[...], w_ref[...],
                         preferred_element_type=jnp.float32) * dinv


def _tc2_body(ap_ref, g_ref, degp_ref, b_ref, w_ref, g2_ref):
    dinv = _dinv_block(degp_ref)
    pre = dinv * (ap_ref[0] + ap_ref[1] + g_ref[...]) + b_ref[...]
    x2 = jnp.maximum(pre, 0.0)
    g2_ref[...] = jnp.dot(x2, w_ref[...],
                          preferred_element_type=jnp.float32) * dinv


def _tc3_body(ap_ref, g_ref, degp_ref, b_ref, out_ref):
    dinv = _dinv_block(degp_ref)
    out_ref[...] = dinv * (ap_ref[0] + ap_ref[1] + g_ref[...]) + b_ref[...]


def _row_spec(r):
    return pl.BlockSpec((r, D), lambda i: (i, 0))


_pair_spec = pl.BlockSpec((NC, R, D), lambda i: (0, i, 0))
_degp_spec = _pair_spec
_full_w = pl.BlockSpec((D, D), lambda i: (0, 0))
_full_b = pl.BlockSpec((1, D), lambda i: (0, 0))
_out_rd = jax.ShapeDtypeStruct((N, D), jnp.float32)

_tc1 = pl.pallas_call(
    _tc1_body,
    grid=(N // R,),
    in_specs=[_row_spec(R), _full_w, _degp_spec],
    out_specs=_row_spec(R),
    out_shape=_out_rd,
)

_tc2 = pl.pallas_call(
    _tc2_body,
    grid=(N // R,),
    in_specs=[_pair_spec, _row_spec(R), _degp_spec, _full_b, _full_w],
    out_specs=_row_spec(R),
    out_shape=_out_rd,
)

_tc3 = pl.pallas_call(
    _tc3_body,
    grid=(N // R,),
    in_specs=[_pair_spec, _row_spec(R), _degp_spec, _full_b],
    out_specs=_row_spec(R),
    out_shape=_out_rd,
)


def kernel(basic_block, edge_index, W1, b1, W2, b2):
    src4 = edge_index[0].reshape(NW, NBLK, CB, K)
    dst4 = edge_index[1].reshape(NW, NBLK, CB, K)
    dst3 = edge_index[1].reshape(NW, NCHUNK, K)
    b1r = b1.reshape(1, D)
    b2r = b2.reshape(1, D)

    degp = _sc_count(dst3)
    g1 = _tc1(basic_block, W1, degp)
    a1p = _sc_scatter(g1, src4, dst4)
    g2 = _tc2(a1p, g1, degp, b1r, W2)
    a2p = _sc_scatter(g2, src4, dst4)
    return _tc3(a2p, g2, degp, b2r)


# speedup vs baseline: 21.7376x; 1.0183x over previous
import functools

import jax
import jax.numpy as jnp
from jax import lax
from jax.experimental import pallas as pl
from jax.experimental.pallas import tpu as pltpu
from jax.experimental.pallas import tpu_sc as plsc

N = 10000
E = 320000
D = 128

NC = 2
NS = 16
NW = NC * NS
EPW = E // NW
K = 80
NCHUNK = EPW // K

NPAD = 10240
RPT = NPAD // NS
RSTAGE = 40
CB = 25
NBLK = NCHUNK // CB
NPAIR = (CB - 1) // 2

_MESH = plsc.VectorSubcoreMesh(core_axis_name="c", subcore_axis_name="s")


def _zero_f32(ref, nrow, ncol):
    z = jnp.zeros((16,), jnp.float32)

    def body(i, _):
        for j in range(ncol // 16):
            ref[i, pl.ds(j * 16, 16)] = z
        return 0

    lax.fori_loop(0, nrow, body, 0, unroll=False)


@functools.partial(
    pl.kernel,
    out_type=jax.ShapeDtypeStruct((NC, NPAD, D), jnp.float32),
    mesh=_MESH,
    scratch_types=[
        pltpu.VMEM((NCHUNK, K), jnp.int32),
        pltpu.VMEM((K, D), jnp.float32),
        pltpu.VMEM((RSTAGE, D), jnp.float32),
        pltpu.VMEM_SHARED((NPAD, D), jnp.float32),
        pltpu.SemaphoreType.DMA,
    ],
)
def _sc_count(dst_hbm, out_hbm, idx_v, ones_v, stage_v, acc_sh, sem):
    c = lax.axis_index("c")
    s = lax.axis_index("s")
    wid = c * NS + s

    pltpu.sync_copy(dst_hbm.at[wid], idx_v)

    one = jnp.ones((16,), jnp.float32)

    def fill_ones(i, _):
        for j in range(D // 16):
            ones_v[i, pl.ds(j * 16, 16)] = one
        return 0

    lax.fori_loop(0, K, fill_ones, 0, unroll=False)

    _zero_f32(stage_v, RSTAGE, D)
    for j in range(RPT // RSTAGE):
        rs = pl.ds(s * RPT + j * RSTAGE, RSTAGE)
        pltpu.make_async_copy(stage_v, acc_sh.at[rs], sem).start()
    for j in range(RPT // RSTAGE):
        rs = pl.ds(s * RPT + j * RSTAGE, RSTAGE)
        pltpu.make_async_copy(stage_v, acc_sh.at[rs], sem).wait()
    plsc.subcore_barrier()

    def body(i, _):
        pltpu.sync_copy(ones_v, acc_sh.at[idx_v.at[i]], add=True)
        return 0

    lax.fori_loop(0, NCHUNK, body, 0, unroll=False)
    plsc.subcore_barrier()

    for j in range(RPT // RSTAGE):
        rs = pl.ds(s * RPT + j * RSTAGE, RSTAGE)
        pltpu.make_async_copy(acc_sh.at[rs], out_hbm.at[c, rs], sem).start()
    for j in range(RPT // RSTAGE):
        rs = pl.ds(s * RPT + j * RSTAGE, RSTAGE)
        pltpu.make_async_copy(acc_sh.at[rs], out_hbm.at[c, rs], sem).wait()


@functools.partial(
    pl.kernel,
    out_type=jax.ShapeDtypeStruct((NC, NPAD, D), jnp.float32),
    mesh=_MESH,
    scratch_types=[
        pltpu.VMEM((CB, K), jnp.int32),
        pltpu.VMEM((CB, K), jnp.int32),
        pltpu.VMEM((K, D), jnp.float32),
        pltpu.VMEM((K, D), jnp.float32),
        pltpu.VMEM((RSTAGE, D), jnp.float32),
        pltpu.VMEM_SHARED((NPAD, D), jnp.float32),
        pltpu.SemaphoreType.DMA,
        pltpu.SemaphoreType.DMA,
    ],
)
def _sc_scatter(g_hbm, src_hbm, dst_hbm, out_hbm,
                src_v, dst_v, rows0, rows1, stage_v, acc_sh, gsem, ssem):
    c = lax.axis_index("c")
    s = lax.axis_index("s")
    wid = c * NS + s

    _zero_f32(stage_v, RSTAGE, D)
    for j in range(RPT // RSTAGE):
        rs = pl.ds(s * RPT + j * RSTAGE, RSTAGE)
        pltpu.make_async_copy(stage_v, acc_sh.at[rs], gsem).start()
    for j in range(RPT // RSTAGE):
        rs = pl.ds(s * RPT + j * RSTAGE, RSTAGE)
        pltpu.make_async_copy(stage_v, acc_sh.at[rs], gsem).wait()
    plsc.subcore_barrier()

    def g_start(i, buf):
        pltpu.make_async_copy(g_hbm.at[src_v.at[i]], buf, gsem).start()

    def g_wait(i, buf):
        pltpu.make_async_copy(g_hbm.at[src_v.at[i]], buf, gsem).wait()

    def s_start(i, buf):
        pltpu.make_async_copy(buf, acc_sh.at[dst_v.at[i]], ssem).start(add=True)

    def s_wait(i, buf):
        pltpu.make_async_copy(buf, acc_sh.at[dst_v.at[i]], ssem).wait()

    for blk in range(NBLK):
        pltpu.sync_copy(src_hbm.at[wid, blk], src_v)
        pltpu.sync_copy(dst_hbm.at[wid, blk], dst_v)

        g_start(0, rows0)
        g_wait(0, rows0)
        g_start(1, rows1)
        s_start(0, rows0)

        def pair(p, _):
            i1 = 2 * p + 1
            g_wait(i1, rows1)
            s_wait(i1 - 1, rows0)
            g_start(i1 + 1, rows0)
            s_start(i1, rows1)
            i2 = 2 * p + 2
            g_wait(i2, rows0)
            s_wait(i1, rows1)

            @pl.when(p < NPAIR - 1)
            def _():
                g_start(i2 + 1, rows1)

            s_start(i2, rows0)
            return 0

        lax.fori_loop(0, NPAIR, pair, 0, unroll=False)
        s_wait(CB - 1, rows0)
    plsc.subcore_barrier()

    for j in range(RPT // RSTAGE):
        rs = pl.ds(s * RPT + j * RSTAGE, RSTAGE)
        pltpu.make_async_copy(acc_sh.at[rs], out_hbm.at[c, rs], gsem).start()
    for j in range(RPT // RSTAGE):
        rs = pl.ds(s * RPT + j * RSTAGE, RSTAGE)
        pltpu.make_async_copy(acc_sh.at[rs], out_hbm.at[c, rs], gsem).wait()


R = 1000


def _dinv_block(degp_ref):
    deg = degp_ref[0, :, 0:1] + degp_ref[1, :, 0:1] + 1.0
    return lax.rsqrt(deg)


def _tc1_body(x_ref, w_ref, degp_ref, g_ref):
    dinv = _dinv_block(degp_ref)
    g_ref[...] = jnp.dot(x_ref[...], w_ref[...],
                         preferred_element_type=jnp.float32) * dinv


def _tc2_body(ap_ref, g_ref, degp_ref, b_ref, w_ref, g2_ref):
    dinv = _dinv_block(degp_ref)
    pre = dinv * (ap_ref[0] + ap_ref[1] + g_ref[...]) + b_ref[...]
    x2 = jnp.maximum(pre, 0.0)
    g2_ref[...] = jnp.dot(x2, w_ref[...],
                          preferred_element_type=jnp.float32) * dinv


def _tc3_body(ap_ref, g_ref, degp_ref, b_ref, out_ref):
    dinv = _dinv_block(degp_ref)
    out_ref[...] = dinv * (ap_ref[0] + ap_ref[1] + g_ref[...]) + b_ref[...]


def _row_spec(r):
    return pl.BlockSpec((r, D), lambda i: (i, 0))


_pair_spec = pl.BlockSpec((NC, R, D), lambda i: (0, i, 0))
_degp_spec = _pair_spec
_full_w = pl.BlockSpec((D, D), lambda i: (0, 0))
_full_b = pl.BlockSpec((1, D), lambda i: (0, 0))
_out_rd = jax.ShapeDtypeStruct((N, D), jnp.float32)

_tc1 = pl.pallas_call(
    _tc1_body,
    grid=(N // R,),
    in_specs=[_row_spec(R), _full_w, _degp_spec],
    out_specs=_row_spec(R),
    out_shape=_out_rd,
)

_tc2 = pl.pallas_call(
    _tc2_body,
    grid=(N // R,),
    in_specs=[_pair_spec, _row_spec(R), _degp_spec, _full_b, _full_w],
    out_specs=_row_spec(R),
    out_shape=_out_rd,
)

_tc3 = pl.pallas_call(
    _tc3_body,
    grid=(N // R,),
    in_specs=[_pair_spec, _row_spec(R), _degp_spec, _full_b],
    out_specs=_row_spec(R),
    out_shape=_out_rd,
)


def kernel(basic_block, edge_index, W1, b1, W2, b2):
    src4 = edge_index[0].reshape(NW, NBLK, CB, K)
    dst4 = edge_index[1].reshape(NW, NBLK, CB, K)
    dst3 = edge_index[1].reshape(NW, NCHUNK, K)
    b1r = b1.reshape(1, D)
    b2r = b2.reshape(1, D)

    degp = _sc_count(dst3)
    g1 = _tc1(basic_block, W1, degp)
    a1p = _sc_scatter(g1, src4, dst4)
    g2 = _tc2(a1p, g1, degp, b1r, W2)
    a2p = _sc_scatter(g2, src4, dst4)
    return _tc3(a2p, g2, degp, b2r)


# speedup vs baseline: 28.1817x; 1.2964x over previous
import functools

import jax
import jax.numpy as jnp
from jax import lax
from jax.experimental import pallas as pl
from jax.experimental.pallas import tpu as pltpu
from jax.experimental.pallas import tpu_sc as plsc

N = 10000
E = 320000
D = 128

NC = 2
NS = 16
NW = NC * NS
EPW = E // NW
K = 80
NCHUNK = EPW // K

NPAD = 10240
RPT = NPAD // NS
RSTAGE = 40
CB = 25
NBLK = NCHUNK // CB
NPAIR = (CB - 1) // 2

_MESH = plsc.VectorSubcoreMesh(core_axis_name="c", subcore_axis_name="s")


def _zero_f32(ref, nrow, ncol):
    z = jnp.zeros((16,), jnp.float32)

    def body(i, _):
        for j in range(ncol // 16):
            ref[i, pl.ds(j * 16, 16)] = z
        return 0

    lax.fori_loop(0, nrow, body, 0, unroll=False)


@functools.partial(
    pl.kernel,
    out_type=jax.ShapeDtypeStruct((NC, NPAD, D), jnp.float32),
    mesh=_MESH,
    scratch_types=[
        pltpu.VMEM((NCHUNK, K), jnp.int32),
        pltpu.VMEM((K, D), jnp.float32),
        pltpu.VMEM((RSTAGE, D), jnp.float32),
        pltpu.VMEM_SHARED((NPAD, D), jnp.float32),
        pltpu.SemaphoreType.DMA,
    ],
)
def _sc_count(dst_hbm, out_hbm, idx_v, ones_v, stage_v, acc_sh, sem):
    c = lax.axis_index("c")
    s = lax.axis_index("s")
    wid = c * NS + s

    pltpu.sync_copy(dst_hbm.at[wid], idx_v)

    one = jnp.ones((16,), jnp.float32)

    def fill_ones(i, _):
        for j in range(D // 16):
            ones_v[i, pl.ds(j * 16, 16)] = one
        return 0

    lax.fori_loop(0, K, fill_ones, 0, unroll=False)

    _zero_f32(stage_v, RSTAGE, D)
    for j in range(RPT // RSTAGE):
        rs = pl.ds(s * RPT + j * RSTAGE, RSTAGE)
        pltpu.make_async_copy(stage_v, acc_sh.at[rs], sem).start()
    for j in range(RPT // RSTAGE):
        rs = pl.ds(s * RPT + j * RSTAGE, RSTAGE)
        pltpu.make_async_copy(stage_v, acc_sh.at[rs], sem).wait()
    plsc.subcore_barrier()

    def body(i, _):
        pltpu.sync_copy(ones_v, acc_sh.at[idx_v.at[i]], add=True)
        return 0

    lax.fori_loop(0, NCHUNK, body, 0, unroll=False)
    plsc.subcore_barrier()

    for j in range(RPT // RSTAGE):
        rs = pl.ds(s * RPT + j * RSTAGE, RSTAGE)
        pltpu.make_async_copy(acc_sh.at[rs], out_hbm.at[c, rs], sem).start()
    for j in range(RPT // RSTAGE):
        rs = pl.ds(s * RPT + j * RSTAGE, RSTAGE)
        pltpu.make_async_copy(acc_sh.at[rs], out_hbm.at[c, rs], sem).wait()


@functools.partial(
    pl.kernel,
    out_type=jax.ShapeDtypeStruct((NC, NPAD, D), jnp.float32),
    mesh=_MESH,
    scratch_types=[
        pltpu.VMEM((CB, K), jnp.int32),
        pltpu.VMEM((CB, K), jnp.int32),
        pltpu.VMEM((K, D), jnp.float32),
        pltpu.VMEM((K, D), jnp.float32),
        pltpu.VMEM((K, D), jnp.float32),
        pltpu.VMEM_SHARED((NPAD, D), jnp.float32),
        pltpu.SemaphoreType.DMA,
        pltpu.SemaphoreType.DMA,
        pltpu.SemaphoreType.DMA,
    ],
)
def _sc_scatter(g_hbm, src_hbm, dst_hbm, out_hbm,
                src_v, dst_v, rows0, rows1, rows2, acc_sh, gsem0, gsem1, ssem):
    c = lax.axis_index("c")
    s = lax.axis_index("s")
    wid = c * NS + s
    bufs = (rows0, rows1, rows2)
    gsems = (gsem0, gsem1)

    _zero_f32(rows0, K, D)
    for j in range(RPT // K):
        rs = pl.ds(s * RPT + j * K, K)
        pltpu.make_async_copy(rows0, acc_sh.at[rs], gsem0).start()
    for j in range(RPT // K):
        rs = pl.ds(s * RPT + j * K, K)
        pltpu.make_async_copy(rows0, acc_sh.at[rs], gsem0).wait()
    plsc.subcore_barrier()

    def g_start(i, buf, gs):
        pltpu.make_async_copy(g_hbm.at[src_v.at[i]], buf, gs).start()

    def g_wait(i, buf, gs):
        pltpu.make_async_copy(g_hbm.at[src_v.at[i]], buf, gs).wait()

    def s_start(i, buf):
        pltpu.make_async_copy(buf, acc_sh.at[dst_v.at[i]], ssem).start(add=True)

    def s_wait(i, buf):
        pltpu.make_async_copy(buf, acc_sh.at[dst_v.at[i]], ssem).wait()

    for blk in range(NBLK):
        pltpu.sync_copy(src_hbm.at[wid, blk], src_v)
        pltpu.sync_copy(dst_hbm.at[wid, blk], dst_v)

        g_start(0, rows0, gsem0)
        g_start(1, rows1, gsem1)

        def six(q, _):
            base = 6 * q
            for u in range(6):
                i = base + u
                buf = bufs[u % 3]
                nbuf = bufs[(u + 2) % 3]
                gs = gsems[u % 2]
                g_wait(i, buf, gs)
                if u == 0:
                    @pl.when(q > 0)
                    def _():
                        s_wait(i - 1, bufs[2])
                else:
                    s_wait(i - 1, bufs[(u - 1) % 3])

                @pl.when(i + 2 < CB)
                def _():
                    g_start(i + 2, nbuf, gs)

                s_start(i, buf)
            return 0

        lax.fori_loop(0, (CB - 1) // 6, six, 0, unroll=False)
        g_wait(CB - 1, rows0, gsem0)
        s_wait(CB - 2, rows2)
        s_start(CB - 1, rows0)
        s_wait(CB - 1, rows0)
    plsc.subcore_barrier()

    for j in range(RPT // RSTAGE):
        rs = pl.ds(s * RPT + j * RSTAGE, RSTAGE)
        pltpu.make_async_copy(acc_sh.at[rs], out_hbm.at[c, rs], gsem0).start()
    for j in range(RPT // RSTAGE):
        rs = pl.ds(s * RPT + j * RSTAGE, RSTAGE)
        pltpu.make_async_copy(acc_sh.at[rs], out_hbm.at[c, rs], gsem0).wait()


R = 1000


def _dinv_block(degp_ref):
    deg = degp_ref[0, :, 0:1] + degp_ref[1, :, 0:1] + 1.0
    return lax.rsqrt(deg)


def _tc1_body(x_ref, w_ref, degp_ref, g_ref):
    dinv = _dinv_block(degp_ref)
    g_ref[...] = jnp.dot(x_ref[...], w_ref[...],
                         preferred_element_type=jnp.float32) * dinv


def _tc2_body(ap_ref, g_ref, degp_ref, b_ref, w_ref, g2_ref):
    dinv = _dinv_block(degp_ref)
    pre = dinv * (ap_ref[0] + ap_ref[1] + g_ref[...]) + b_ref[...]
    x2 = jnp.maximum(pre, 0.0)
    g2_ref[...] = jnp.dot(x2, w_ref[...],
                          preferred_element_type=jnp.float32) * dinv


def _tc3_body(ap_ref, g_ref, degp_ref, b_ref, out_ref):
    dinv = _dinv_block(degp_ref)
    out_ref[...] = dinv * (ap_ref[0] + ap_ref[1] + g_ref[...]) + b_ref[...]


def _row_spec(r):
    return pl.BlockSpec((r, D), lambda i: (i, 0))


_pair_spec = pl.BlockSpec((NC, R, D), lambda i: (0, i, 0))
_degp_spec = _pair_spec
_full_w = pl.BlockSpec((D, D), lambda i: (0, 0))
_full_b = pl.BlockSpec((1, D), lambda i: (0, 0))
_out_rd = jax.ShapeDtypeStruct((N, D), jnp.float32)

_tc1 = pl.pallas_call(
    _tc1_body,
    grid=(N // R,),
    in_specs=[_row_spec(R), _full_w, _degp_spec],
    out_specs=_row_spec(R),
    out_shape=_out_rd,
)

_tc2 = pl.pallas_call(
    _tc2_body,
    grid=(N // R,),
    in_specs=[_pair_spec, _row_spec(R), _degp_spec, _full_b, _full_w],
    out_specs=_row_spec(R),
    out_shape=_out_rd,
)

_tc3 = pl.pallas_call(
    _tc3_body,
    grid=(N // R,),
    in_specs=[_pair_spec, _row_spec(R), _degp_spec, _full_b],
    out_specs=_row_spec(R),
    out_shape=_out_rd,
)


def kernel(basic_block, edge_index, W1, b1, W2, b2):
    src4 = edge_index[0].reshape(NW, NBLK, CB, K)
    dst4 = edge_index[1].reshape(NW, NBLK, CB, K)
    dst3 = edge_index[1].reshape(NW, NCHUNK, K)
    b1r = b1.reshape(1, D)
    b2r = b2.reshape(1, D)

    degp = _sc_count(dst3)
    g1 = _tc1(basic_block, W1, degp)
    a1p = _sc_scatter(g1, src4, dst4)
    g2 = _tc2(a1p, g1, degp, b1r, W2)
    a2p = _sc_scatter(g2, src4, dst4)
    return _tc3(a2p, g2, degp, b2r)


# speedup vs baseline: 28.3439x; 1.0058x over previous
import functools

import jax
import jax.numpy as jnp
from jax import lax
from jax.experimental import pallas as pl
from jax.experimental.pallas import tpu as pltpu
from jax.experimental.pallas import tpu_sc as plsc

N = 10000
E = 320000
D = 128

NC = 2
NS = 16
NW = NC * NS
EPW = E // NW
K = 80
NCHUNK = EPW // K

NPAD = 10240
RPT = NPAD // NS
RSTAGE = 40
CB = 25
NBLK = NCHUNK // CB
NPAIR = (CB - 1) // 2

_MESH = plsc.VectorSubcoreMesh(core_axis_name="c", subcore_axis_name="s")


def _zero_f32(ref, nrow, ncol):
    z = jnp.zeros((16,), jnp.float32)

    def body(i, _):
        for j in range(ncol // 16):
            ref[i, pl.ds(j * 16, 16)] = z
        return 0

    lax.fori_loop(0, nrow, body, 0, unroll=False)


@functools.partial(
    pl.kernel,
    out_type=jax.ShapeDtypeStruct((NC, NPAD, D), jnp.float32),
    mesh=_MESH,
    scratch_types=[
        pltpu.VMEM((NCHUNK, K), jnp.int32),
        pltpu.VMEM((K, D), jnp.float32),
        pltpu.VMEM((RSTAGE, D), jnp.float32),
        pltpu.VMEM_SHARED((NPAD, D), jnp.float32),
        pltpu.SemaphoreType.DMA,
        pltpu.SemaphoreType.DMA,
    ],
)
def _sc_count(dst_hbm, out_hbm, idx_v, ones_v, stage_v, acc_sh, sem, sem2):
    c = lax.axis_index("c")
    s = lax.axis_index("s")
    wid = c * NS + s

    pltpu.sync_copy(dst_hbm.at[wid], idx_v)

    one = jnp.ones((16,), jnp.float32)

    def fill_ones(i, _):
        for j in range(D // 16):
            ones_v[i, pl.ds(j * 16, 16)] = one
        return 0

    lax.fori_loop(0, K, fill_ones, 0, unroll=False)

    _zero_f32(stage_v, RSTAGE, D)
    for j in range(RPT // RSTAGE):
        rs = pl.ds(s * RPT + j * RSTAGE, RSTAGE)
        pltpu.make_async_copy(stage_v, acc_sh.at[rs], sem).start()
    for j in range(RPT // RSTAGE):
        rs = pl.ds(s * RPT + j * RSTAGE, RSTAGE)
        pltpu.make_async_copy(stage_v, acc_sh.at[rs], sem).wait()
    plsc.subcore_barrier()

    def s_start(i, sm):
        pltpu.make_async_copy(ones_v, acc_sh.at[idx_v.at[i]], sm).start(
            add=True)

    def s_wait(i, sm):
        pltpu.make_async_copy(ones_v, acc_sh.at[idx_v.at[i]], sm).wait()

    s_start(0, sem)

    def body(p, _):
        i = 2 * p
        s_start(i + 1, sem2)
        s_wait(i, sem)
        @pl.when(i + 2 < NCHUNK)
        def _():
            s_start(i + 2, sem)
        s_wait(i + 1, sem2)
        return 0

    lax.fori_loop(0, NCHUNK // 2, body, 0, unroll=False)
    s_wait(NCHUNK - 1, sem)
    plsc.subcore_barrier()

    for j in range(RPT // RSTAGE):
        rs = pl.ds(s * RPT + j * RSTAGE, RSTAGE)
        pltpu.make_async_copy(acc_sh.at[rs], out_hbm.at[c, rs], sem).start()
    for j in range(RPT // RSTAGE):
        rs = pl.ds(s * RPT + j * RSTAGE, RSTAGE)
        pltpu.make_async_copy(acc_sh.at[rs], out_hbm.at[c, rs], sem).wait()


@functools.partial(
    pl.kernel,
    out_type=jax.ShapeDtypeStruct((NC, NPAD, D), jnp.float32),
    mesh=_MESH,
    scratch_types=[
        pltpu.VMEM((CB, K), jnp.int32),
        pltpu.VMEM((CB, K), jnp.int32),
        pltpu.VMEM((K, D), jnp.float32),
        pltpu.VMEM((K, D), jnp.float32),
        pltpu.VMEM((K, D), jnp.float32),
        pltpu.VMEM_SHARED((NPAD, D), jnp.float32),
        pltpu.SemaphoreType.DMA,
        pltpu.SemaphoreType.DMA,
        pltpu.SemaphoreType.DMA,
    ],
)
def _sc_scatter(g_hbm, src_hbm, dst_hbm, out_hbm,
                src_v, dst_v, rows0, rows1, rows2, acc_sh, gsem0, gsem1, ssem):
    c = lax.axis_index("c")
    s = lax.axis_index("s")
    wid = c * NS + s
    bufs = (rows0, rows1, rows2)
    gsems = (gsem0, gsem1)

    _zero_f32(rows0, K, D)
    for j in range(RPT // K):
        rs = pl.ds(s * RPT + j * K, K)
        pltpu.make_async_copy(rows0, acc_sh.at[rs], gsem0).start()
    for j in range(RPT // K):
        rs = pl.ds(s * RPT + j * K, K)
        pltpu.make_async_copy(rows0, acc_sh.at[rs], gsem0).wait()
    plsc.subcore_barrier()

    def g_start(i, buf, gs):
        pltpu.make_async_copy(g_hbm.at[src_v.at[i]], buf, gs).start()

    def g_wait(i, buf, gs):
        pltpu.make_async_copy(g_hbm.at[src_v.at[i]], buf, gs).wait()

    def s_start(i, buf):
        pltpu.make_async_copy(buf, acc_sh.at[dst_v.at[i]], ssem).start(add=True)

    def s_wait(i, buf):
        pltpu.make_async_copy(buf, acc_sh.at[dst_v.at[i]], ssem).wait()

    for blk in range(NBLK):
        pltpu.sync_copy(src_hbm.at[wid, blk], src_v)
        pltpu.sync_copy(dst_hbm.at[wid, blk], dst_v)

        g_start(0, rows0, gsem0)
        g_start(1, rows1, gsem1)

        def six(q, _):
            base = 6 * q
            for u in range(6):
                i = base + u
                buf = bufs[u % 3]
                nbuf = bufs[(u + 2) % 3]
                gs = gsems[u % 2]
                g_wait(i, buf, gs)
                if u == 0:
                    @pl.when(q > 0)
                    def _():
                        s_wait(i - 1, bufs[2])
                else:
                    s_wait(i - 1, bufs[(u - 1) % 3])

                @pl.when(i + 2 < CB)
                def _():
                    g_start(i + 2, nbuf, gs)

                s_start(i, buf)
            return 0

        lax.fori_loop(0, (CB - 1) // 6, six, 0, unroll=False)
        g_wait(CB - 1, rows0, gsem0)
        s_wait(CB - 2, rows2)
        s_start(CB - 1, rows0)
        s_wait(CB - 1, rows0)
    plsc.subcore_barrier()

    for j in range(RPT // RSTAGE):
        rs = pl.ds(s * RPT + j * RSTAGE, RSTAGE)
        pltpu.make_async_copy(acc_sh.at[rs], out_hbm.at[c, rs], gsem0).start()
    for j in range(RPT // RSTAGE):
        rs = pl.ds(s * RPT + j * RSTAGE, RSTAGE)
        pltpu.make_async_copy(acc_sh.at[rs], out_hbm.at[c, rs], gsem0).wait()


R = 1000


def _dinv_block(degp_ref):
    deg = degp_ref[0, :, 0:1] + degp_ref[1, :, 0:1] + 1.0
    return lax.rsqrt(deg)


def _tc1_body(x_ref, w_ref, degp_ref, g_ref):
    dinv = _dinv_block(degp_ref)
    g_ref[...] = jnp.dot(x_ref[...], w_ref[...],
                         preferred_element_type=jnp.float32) * dinv


def _tc2_body(ap_ref, g_ref, degp_ref, b_ref, w_ref, g2_ref):
    dinv = _dinv_block(degp_ref)
    pre = dinv * (ap_ref[0] + ap_ref[1] + g_ref[...]) + b_ref[...]
    x2 = jnp.maximum(pre, 0.0)
    g2_ref[...] = jnp.dot(x2, w_ref[...],
                          preferred_element_type=jnp.float32) * dinv


def _tc3_body(ap_ref, g_ref, degp_ref, b_ref, out_ref):
    dinv = _dinv_block(degp_ref)
    out_ref[...] = dinv * (ap_ref[0] + ap_ref[1] + g_ref[...]) + b_ref[...]


def _row_spec(r):
    return pl.BlockSpec((r, D), lambda i: (i, 0))


_pair_spec = pl.BlockSpec((NC, R, D), lambda i: (0, i, 0))
_degp_spec = _pair_spec
_full_w = pl.BlockSpec((D, D), lambda i: (0, 0))
_full_b = pl.BlockSpec((1, D), lambda i: (0, 0))
_out_rd = jax.ShapeDtypeStruct((N, D), jnp.float32)

_tc1 = pl.pallas_call(
    _tc1_body,
    grid=(N // R,),
    in_specs=[_row_spec(R), _full_w, _degp_spec],
    out_specs=_row_spec(R),
    out_shape=_out_rd,
)

_tc2 = pl.pallas_call(
    _tc2_body,
    grid=(N // R,),
    in_specs=[_pair_spec, _row_spec(R), _degp_spec, _full_b, _full_w],
    out_specs=_row_spec(R),
    out_shape=_out_rd,
)

_tc3 = pl.pallas_call(
    _tc3_body,
    grid=(N // R,),
    in_specs=[_pair_spec, _row_spec(R), _degp_spec, _full_b],
    out_specs=_row_spec(R),
    out_shape=_out_rd,
)


def kernel(basic_block, edge_index, W1, b1, W2, b2):
    src4 = edge_index[0].reshape(NW, NBLK, CB, K)
    dst4 = edge_index[1].reshape(NW, NBLK, CB, K)
    dst3 = edge_index[1].reshape(NW, NCHUNK, K)
    b1r = b1.reshape(1, D)
    b2r = b2.reshape(1, D)

    degp = _sc_count(dst3)
    g1 = _tc1(basic_block, W1, degp)
    a1p = _sc_scatter(g1, src4, dst4)
    g2 = _tc2(a1p, g1, degp, b1r, W2)
    a2p = _sc_scatter(g2, src4, dst4)
    return _tc3(a2p, g2, degp, b2r)


# speedup vs baseline: 28.8204x; 1.0168x over previous
import functools

import jax
import jax.numpy as jnp
from jax import lax
from jax.experimental import pallas as pl
from jax.experimental.pallas import tpu as pltpu
from jax.experimental.pallas import tpu_sc as plsc

N = 10000
E = 320000
D = 128

NC = 2
NS = 16
NW = NC * NS
EPW = E // NW
K = 80
NCHUNK = EPW // K

NPAD = 10240
RPT = NPAD // NS
RSTAGE = 40
CB = 25
NBLK = NCHUNK // CB
NPAIR = (CB - 1) // 2

_MESH = plsc.VectorSubcoreMesh(core_axis_name="c", subcore_axis_name="s")


def _zero_f32(ref, nrow, ncol):
    z = jnp.zeros((16,), jnp.float32)

    def body(i, _):
        for j in range(ncol // 16):
            ref[i, pl.ds(j * 16, 16)] = z
        return 0

    lax.fori_loop(0, nrow, body, 0, unroll=False)


@functools.partial(
    pl.kernel,
    out_type=jax.ShapeDtypeStruct((NC, NPAD, D), jnp.float32),
    mesh=_MESH,
    scratch_types=[
        pltpu.VMEM((NCHUNK, K), jnp.int32),
        pltpu.VMEM((K, D), jnp.float32),
        pltpu.VMEM((RSTAGE, D), jnp.float32),
        pltpu.VMEM_SHARED((NPAD, D), jnp.float32),
        pltpu.SemaphoreType.DMA,
        pltpu.SemaphoreType.DMA,
    ],
)
def _sc_count(dst_hbm, out_hbm, idx_v, ones_v, stage_v, acc_sh, sem, sem2):
    c = lax.axis_index("c")
    s = lax.axis_index("s")
    wid = c * NS + s

    pltpu.sync_copy(dst_hbm.at[wid], idx_v)

    one = jnp.ones((16,), jnp.float32)

    def fill_ones(i, _):
        for j in range(D // 16):
            ones_v[i, pl.ds(j * 16, 16)] = one
        return 0

    lax.fori_loop(0, K, fill_ones, 0, unroll=False)

    _zero_f32(stage_v, RSTAGE, D)
    for j in range(RPT // RSTAGE):
        rs = pl.ds(s * RPT + j * RSTAGE, RSTAGE)
        pltpu.make_async_copy(stage_v, acc_sh.at[rs], sem).start()
    for j in range(RPT // RSTAGE):
        rs = pl.ds(s * RPT + j * RSTAGE, RSTAGE)
        pltpu.make_async_copy(stage_v, acc_sh.at[rs], sem).wait()
    plsc.subcore_barrier()

    def s_start(i, sm):
        pltpu.make_async_copy(ones_v, acc_sh.at[idx_v.at[i]], sm).start(
            add=True)

    def s_wait(i, sm):
        pltpu.make_async_copy(ones_v, acc_sh.at[idx_v.at[i]], sm).wait()

    s_start(0, sem)

    def body(p, _):
        i = 2 * p
        s_start(i + 1, sem2)
        s_wait(i, sem)
        @pl.when(i + 2 < NCHUNK)
        def _():
            s_start(i + 2, sem)
        s_wait(i + 1, sem2)
        return 0

    lax.fori_loop(0, NCHUNK // 2, body, 0, unroll=False)
    s_wait(NCHUNK - 1, sem)
    plsc.subcore_barrier()

    for j in range(RPT // RSTAGE):
        rs = pl.ds(s * RPT + j * RSTAGE, RSTAGE)
        pltpu.make_async_copy(acc_sh.at[rs], out_hbm.at[c, rs], sem).start()
    for j in range(RPT // RSTAGE):
        rs = pl.ds(s * RPT + j * RSTAGE, RSTAGE)
        pltpu.make_async_copy(acc_sh.at[rs], out_hbm.at[c, rs], sem).wait()


@functools.partial(
    pl.kernel,
    out_type=jax.ShapeDtypeStruct((NC, NPAD, D), jnp.float32),
    mesh=_MESH,
    scratch_types=[
        pltpu.VMEM((2, CB, K), jnp.int32),
        pltpu.VMEM((K, D), jnp.float32),
        pltpu.VMEM((K, D), jnp.float32),
        pltpu.VMEM((K, D), jnp.float32),
        pltpu.VMEM_SHARED((NPAD, D), jnp.float32),
        pltpu.SemaphoreType.DMA,
        pltpu.SemaphoreType.DMA,
        pltpu.SemaphoreType.DMA,
    ],
)
def _sc_scatter(g_hbm, idx_hbm, out_hbm,
                idx_v, rows0, rows1, rows2, acc_sh, gsem0, gsem1, ssem):
    c = lax.axis_index("c")
    s = lax.axis_index("s")
    wid = c * NS + s
    bufs = (rows0, rows1, rows2)
    gsems = (gsem0, gsem1)

    _zero_f32(rows0, K, D)
    for j in range(RPT // K):
        rs = pl.ds(s * RPT + j * K, K)
        pltpu.make_async_copy(rows0, acc_sh.at[rs], gsem0).start()
    for j in range(RPT // K):
        rs = pl.ds(s * RPT + j * K, K)
        pltpu.make_async_copy(rows0, acc_sh.at[rs], gsem0).wait()
    plsc.subcore_barrier()

    def g_start(i, buf, gs):
        pltpu.make_async_copy(g_hbm.at[idx_v.at[0, i]], buf, gs).start()

    def g_wait(i, buf, gs):
        pltpu.make_async_copy(g_hbm.at[idx_v.at[0, i]], buf, gs).wait()

    def s_start(i, buf):
        pltpu.make_async_copy(buf, acc_sh.at[idx_v.at[1, i]],
                              ssem).start(add=True)

    def s_wait(i, buf):
        pltpu.make_async_copy(buf, acc_sh.at[idx_v.at[1, i]], ssem).wait()

    for blk in range(NBLK):
        pltpu.sync_copy(idx_hbm.at[wid, blk], idx_v)

        g_start(0, rows0, gsem0)
        g_start(1, rows1, gsem1)

        def six(q, _):
            base = 6 * q
            for u in range(6):
                i = base + u
                buf = bufs[u % 3]
                nbuf = bufs[(u + 2) % 3]
                gs = gsems[u % 2]
                g_wait(i, buf, gs)
                if u == 0:
                    @pl.when(q > 0)
                    def _():
                        s_wait(i - 1, bufs[2])
                else:
                    s_wait(i - 1, bufs[(u - 1) % 3])

                @pl.when(i + 2 < CB)
                def _():
                    g_start(i + 2, nbuf, gs)

                s_start(i, buf)
            return 0

        lax.fori_loop(0, (CB - 1) // 6, six, 0, unroll=False)
        g_wait(CB - 1, rows0, gsem0)
        s_wait(CB - 2, rows2)
        s_start(CB - 1, rows0)
        s_wait(CB - 1, rows0)
    plsc.subcore_barrier()

    for j in range(RPT // RSTAGE):
        rs = pl.ds(s * RPT + j * RSTAGE, RSTAGE)
        pltpu.make_async_copy(acc_sh.at[rs], out_hbm.at[c, rs], gsem0).start()
    for j in range(RPT // RSTAGE):
        rs = pl.ds(s * RPT + j * RSTAGE, RSTAGE)
        pltpu.make_async_copy(acc_sh.at[rs], out_hbm.at[c, rs], gsem0).wait()


R = 1000


def _dinv_block(degp_ref):
    deg = degp_ref[0, :, 0:1] + degp_ref[1, :, 0:1] + 1.0
    return lax.rsqrt(deg)


def _tc1_body(x_ref, w_ref, degp_ref, g_ref):
    dinv = _dinv_block(degp_ref)
    g_ref[...] = jnp.dot(x_ref[...], w_ref[...],
                         preferred_element_type=jnp.float32) * dinv


def _tc2_body(ap_ref, g_ref, degp_ref, b_ref, w_ref, g2_ref):
    dinv = _dinv_block(degp_ref)
    pre = dinv * (ap_ref[0] + ap_ref[1] + g_ref[...]) + b_ref[...]
    x2 = jnp.maximum(pre, 0.0)
    g2_ref[...] = jnp.dot(x2, w_ref[...],
                          preferred_element_type=jnp.float32) * dinv


def _tc3_body(ap_ref, g_ref, degp_ref, b_ref, out_ref):
    dinv = _dinv_block(degp_ref)
    out_ref[...] = dinv * (ap_ref[0] + ap_ref[1] + g_ref[...]) + b_ref[...]


def _row_spec(r):
    return pl.BlockSpec((r, D), lambda i: (i, 0))


_pair_spec = pl.BlockSpec((NC, R, D), lambda i: (0, i, 0))
_degp_spec = _pair_spec
_full_w = pl.BlockSpec((D, D), lambda i: (0, 0))
_full_b = pl.BlockSpec((1, D), lambda i: (0, 0))
_out_rd = jax.ShapeDtypeStruct((N, D), jnp.float32)

_tc1 = pl.pallas_call(
    _tc1_body,
    grid=(N // R,),
    in_specs=[_row_spec(R), _full_w, _degp_spec],
    out_specs=_row_spec(R),
    out_shape=_out_rd,
)

_tc2 = pl.pallas_call(
    _tc2_body,
    grid=(N // R,),
    in_specs=[_pair_spec, _row_spec(R), _degp_spec, _full_b, _full_w],
    out_specs=_row_spec(R),
    out_shape=_out_rd,
)

_tc3 = pl.pallas_call(
    _tc3_body,
    grid=(N // R,),
    in_specs=[_pair_spec, _row_spec(R), _degp_spec, _full_b],
    out_specs=_row_spec(R),
    out_shape=_out_rd,
)


def kernel(basic_block, edge_index, W1, b1, W2, b2):
    ei4 = edge_index.reshape(2, NW, NBLK, CB, K)
    idx5 = ei4.transpose(1, 2, 0, 3, 4)
    dst3 = edge_index[1].reshape(NW, NCHUNK, K)
    b1r = b1.reshape(1, D)
    b2r = b2.reshape(1, D)

    degp = _sc_count(dst3)
    g1 = _tc1(basic_block, W1, degp)
    a1p = _sc_scatter(g1, idx5)
    g2 = _tc2(a1p, g1, degp, b1r, W2)
    a2p = _sc_scatter(g2, idx5)
    return _tc3(a2p, g2, degp, b2r)


# speedup vs baseline: 29.1388x; 1.0110x over previous
import functools

import jax
import jax.numpy as jnp
from jax import lax
from jax.experimental import pallas as pl
from jax.experimental.pallas import tpu as pltpu
from jax.experimental.pallas import tpu_sc as plsc

N = 10000
E = 320000
D = 128

NC = 2
NS = 16
NW = NC * NS
EPW = E // NW
K = 80
NCHUNK = EPW // K

NPAD = 10240
RPT = NPAD // NS
RSTAGE = 40
CB = 25
NBLK = NCHUNK // CB
NPAIR = (CB - 1) // 2

_MESH = plsc.VectorSubcoreMesh(core_axis_name="c", subcore_axis_name="s")


def _zero_f32(ref, nrow, ncol):
    z = jnp.zeros((16,), jnp.float32)

    def body(i, _):
        for j in range(ncol // 16):
            ref[i, pl.ds(j * 16, 16)] = z
        return 0

    lax.fori_loop(0, nrow, body, 0, unroll=False)


@functools.partial(
    pl.kernel,
    out_type=jax.ShapeDtypeStruct((NC, NPAD, D), jnp.float32),
    mesh=_MESH,
    scratch_types=[
        pltpu.VMEM((NCHUNK, K), jnp.int32),
        pltpu.VMEM((K, D), jnp.float32),
        pltpu.VMEM((RSTAGE, D), jnp.float32),
        pltpu.VMEM_SHARED((NPAD, D), jnp.float32),
        pltpu.SemaphoreType.DMA,
        pltpu.SemaphoreType.DMA,
    ],
)
def _sc_count(dst_hbm, out_hbm, idx_v, ones_v, stage_v, acc_sh, sem, sem2):
    c = lax.axis_index("c")
    s = lax.axis_index("s")
    wid = c * NS + s

    pltpu.sync_copy(dst_hbm.at[wid], idx_v)

    one = jnp.ones((16,), jnp.float32)

    def fill_ones(i, _):
        for j in range(D // 16):
            ones_v[i, pl.ds(j * 16, 16)] = one
        return 0

    lax.fori_loop(0, K, fill_ones, 0, unroll=False)

    _zero_f32(stage_v, RSTAGE, D)
    for j in range(RPT // RSTAGE):
        rs = pl.ds(s * RPT + j * RSTAGE, RSTAGE)
        pltpu.make_async_copy(stage_v, acc_sh.at[rs], sem).start()
    for j in range(RPT // RSTAGE):
        rs = pl.ds(s * RPT + j * RSTAGE, RSTAGE)
        pltpu.make_async_copy(stage_v, acc_sh.at[rs], sem).wait()
    plsc.subcore_barrier()

    def s_start(i, sm):
        pltpu.make_async_copy(ones_v, acc_sh.at[idx_v.at[i]], sm).start(
            add=True)

    def s_wait(i, sm):
        pltpu.make_async_copy(ones_v, acc_sh.at[idx_v.at[i]], sm).wait()

    s_start(0, sem)

    def body(p, _):
        i = 2 * p
        s_start(i + 1, sem2)
        s_wait(i, sem)
        @pl.when(i + 2 < NCHUNK)
        def _():
            s_start(i + 2, sem)
        s_wait(i + 1, sem2)
        return 0

    lax.fori_loop(0, NCHUNK // 2, body, 0, unroll=False)
    s_wait(NCHUNK - 1, sem)
    plsc.subcore_barrier()

    for j in range(RPT // RSTAGE):
        rs = pl.ds(s * RPT + j * RSTAGE, RSTAGE)
        pltpu.make_async_copy(acc_sh.at[rs], out_hbm.at[c, rs], sem).start()
    for j in range(RPT // RSTAGE):
        rs = pl.ds(s * RPT + j * RSTAGE, RSTAGE)
        pltpu.make_async_copy(acc_sh.at[rs], out_hbm.at[c, rs], sem).wait()


@functools.partial(
    pl.kernel,
    out_type=jax.ShapeDtypeStruct((NC, NPAD, D), jnp.float32),
    mesh=_MESH,
    scratch_types=[
        pltpu.VMEM((2, 2, CB, K), jnp.int32),
        pltpu.VMEM((K, D), jnp.float32),
        pltpu.VMEM((K, D), jnp.float32),
        pltpu.VMEM((K, D), jnp.float32),
        pltpu.VMEM_SHARED((NPAD, D), jnp.float32),
        pltpu.SemaphoreType.DMA,
        pltpu.SemaphoreType.DMA,
        pltpu.SemaphoreType.DMA,
        pltpu.SemaphoreType.DMA,
    ],
)
def _sc_scatter(g_hbm, idx_hbm, out_hbm,
                idx_v, rows0, rows1, rows2, acc_sh, gsem0, gsem1, ssem, isem):
    c = lax.axis_index("c")
    s = lax.axis_index("s")
    wid = c * NS + s
    bufs = (rows0, rows1, rows2)
    gsems = (gsem0, gsem1)

    _zero_f32(rows0, K, D)
    for j in range(RPT // K):
        rs = pl.ds(s * RPT + j * K, K)
        pltpu.make_async_copy(rows0, acc_sh.at[rs], gsem0).start()
    for j in range(RPT // K):
        rs = pl.ds(s * RPT + j * K, K)
        pltpu.make_async_copy(rows0, acc_sh.at[rs], gsem0).wait()
    plsc.subcore_barrier()

    def g_start(bb, i, buf, gs):
        pltpu.make_async_copy(g_hbm.at[idx_v.at[bb, 0, i]], buf, gs).start()

    def g_wait(bb, i, buf, gs):
        pltpu.make_async_copy(g_hbm.at[idx_v.at[bb, 0, i]], buf, gs).wait()

    def s_start(bb, i, buf):
        pltpu.make_async_copy(buf, acc_sh.at[idx_v.at[bb, 1, i]],
                              ssem).start(add=True)

    def s_wait(bb, i, buf):
        pltpu.make_async_copy(buf, acc_sh.at[idx_v.at[bb, 1, i]], ssem).wait()

    pltpu.sync_copy(idx_hbm.at[wid, 0], idx_v.at[0])
    for blk in range(NBLK):
        bb = blk % 2

        g_start(bb, 0, rows0, gsem0)
        g_start(bb, 1, rows1, gsem1)
        if blk + 1 < NBLK:
            pltpu.make_async_copy(idx_hbm.at[wid, blk + 1],
                                  idx_v.at[1 - bb], isem).start()

        def six(q, _):
            base = 6 * q
            for u in range(6):
                i = base + u
                buf = bufs[u % 3]
                nbuf = bufs[(u + 2) % 3]
                gs = gsems[u % 2]
                g_wait(bb, i, buf, gs)
                if u == 0:
                    @pl.when(q > 0)
                    def _():
                        s_wait(bb, i - 1, bufs[2])
                else:
                    s_wait(bb, i - 1, bufs[(u - 1) % 3])

                @pl.when(i + 2 < CB)
                def _():
                    g_start(bb, i + 2, nbuf, gs)

                s_start(bb, i, buf)
            return 0

        lax.fori_loop(0, (CB - 1) // 6, six, 0, unroll=False)
        g_wait(bb, CB - 1, rows0, gsem0)
        s_wait(bb, CB - 2, rows2)
        s_start(bb, CB - 1, rows0)
        s_wait(bb, CB - 1, rows0)
        if blk + 1 < NBLK:
            pltpu.make_async_copy(idx_hbm.at[wid, blk + 1],
                                  idx_v.at[1 - bb], isem).wait()
    plsc.subcore_barrier()

    for j in range(RPT // RSTAGE):
        rs = pl.ds(s * RPT + j * RSTAGE, RSTAGE)
        pltpu.make_async_copy(acc_sh.at[rs], out_hbm.at[c, rs], gsem0).start()
    for j in range(RPT // RSTAGE):
        rs = pl.ds(s * RPT + j * RSTAGE, RSTAGE)
        pltpu.make_async_copy(acc_sh.at[rs], out_hbm.at[c, rs], gsem0).wait()


R = 1000


def _dinv_block(degp_ref):
    deg = degp_ref[0, :, 0:1] + degp_ref[1, :, 0:1] + 1.0
    return lax.rsqrt(deg)


def _tc1_body(x_ref, w_ref, degp_ref, g_ref):
    dinv = _dinv_block(degp_ref)
    g_ref[...] = jnp.dot(x_ref[...], w_ref[...],
                         preferred_element_type=jnp.float32) * dinv


def _tc2_body(ap_ref, g_ref, degp_ref, b_ref, w_ref, g2_ref):
    dinv = _dinv_block(degp_ref)
    pre = dinv * (ap_ref[0] + ap_ref[1] + g_ref[...]) + b_ref[...]
    x2 = jnp.maximum(pre, 0.0)
    g2_ref[...] = jnp.dot(x2, w_ref[...],
                          preferred_element_type=jnp.float32) * dinv


def _tc3_body(ap_ref, g_ref, degp_ref, b_ref, out_ref):
    dinv = _dinv_block(degp_ref)
    out_ref[...] = dinv * (ap_ref[0] + ap_ref[1] + g_ref[...]) + b_ref[...]


def _row_spec(r):
    return pl.BlockSpec((r, D), lambda i: (i, 0))


_pair_spec = pl.BlockSpec((NC, R, D), lambda i: (0, i, 0))
_degp_spec = _pair_spec
_full_w = pl.BlockSpec((D, D), lambda i: (0, 0))
_full_b = pl.BlockSpec((1, D), lambda i: (0, 0))
_out_rd = jax.ShapeDtypeStruct((N, D), jnp.float32)

_tc1 = pl.pallas_call(
    _tc1_body,
    grid=(N // R,),
    in_specs=[_row_spec(R), _full_w, _degp_spec],
    out_specs=_row_spec(R),
    out_shape=_out_rd,
)

_tc2 = pl.pallas_call(
    _tc2_body,
    grid=(N // R,),
    in_specs=[_pair_spec, _row_spec(R), _degp_spec, _full_b, _full_w],
    out_specs=_row_spec(R),
    out_shape=_out_rd,
)

_tc3 = pl.pallas_call(
    _tc3_body,
    grid=(N // R,),
    in_specs=[_pair_spec, _row_spec(R), _degp_spec, _full_b],
    out_specs=_row_spec(R),
    out_shape=_out_rd,
)


def kernel(basic_block, edge_index, W1, b1, W2, b2):
    ei4 = edge_index.reshape(2, NW, NBLK, CB, K)
    idx5 = ei4.transpose(1, 2, 0, 3, 4)
    dst3 = edge_index[1].reshape(NW, NCHUNK, K)
    b1r = b1.reshape(1, D)
    b2r = b2.reshape(1, D)

    degp = _sc_count(dst3)
    g1 = _tc1(basic_block, W1, degp)
    a1p = _sc_scatter(g1, idx5)
    g2 = _tc2(a1p, g1, degp, b1r, W2)
    a2p = _sc_scatter(g2, idx5)
    return _tc3(a2p, g2, degp, b2r)


# speedup vs baseline: 29.1533x; 1.0005x over previous
import functools

import jax
import jax.numpy as jnp
from jax import lax
from jax.experimental import pallas as pl
from jax.experimental.pallas import tpu as pltpu
from jax.experimental.pallas import tpu_sc as plsc

N = 10000
E = 320000
D = 128

NC = 2
NS = 16
NW = NC * NS
EPW = E // NW
K = 80
NCHUNK = EPW // K

NPAD = 10240
RPT = NPAD // NS
RSTAGE = 40
CB = 25
NBLK = NCHUNK // CB
NPAIR = (CB - 1) // 2

_MESH = plsc.VectorSubcoreMesh(core_axis_name="c", subcore_axis_name="s")


def _zero_f32(ref, nrow, ncol):
    z = jnp.zeros((16,), jnp.float32)

    def body(i, _):
        for j in range(ncol // 16):
            ref[i, pl.ds(j * 16, 16)] = z
        return 0

    lax.fori_loop(0, nrow, body, 0, unroll=False)


@functools.partial(
    pl.kernel,
    out_type=jax.ShapeDtypeStruct((NC, NPAD, D), jnp.float32),
    mesh=_MESH,
    scratch_types=[
        pltpu.VMEM((NCHUNK, K), jnp.int32),
        pltpu.VMEM((K, D), jnp.float32),
        pltpu.VMEM((RSTAGE, D), jnp.float32),
        pltpu.VMEM_SHARED((NPAD, D), jnp.float32),
        pltpu.SemaphoreType.DMA,
    ],
)
def _sc_count(dst_hbm, out_hbm, idx_v, ones_v, stage_v, acc_sh, sem):
    c = lax.axis_index("c")
    s = lax.axis_index("s")
    wid = c * NS + s

    pltpu.sync_copy(dst_hbm.at[wid], idx_v)

    one = jnp.ones((16,), jnp.float32)

    def fill_ones(i, _):
        for j in range(D // 16):
            ones_v[i, pl.ds(j * 16, 16)] = one
        return 0

    lax.fori_loop(0, K, fill_ones, 0, unroll=False)

    _zero_f32(stage_v, RSTAGE, D)
    for j in range(RPT // RSTAGE):
        rs = pl.ds(s * RPT + j * RSTAGE, RSTAGE)
        pltpu.make_async_copy(stage_v, acc_sh.at[rs], sem).start()
    for j in range(RPT // RSTAGE):
        rs = pl.ds(s * RPT + j * RSTAGE, RSTAGE)
        pltpu.make_async_copy(stage_v, acc_sh.at[rs], sem).wait()
    plsc.subcore_barrier()

    WIN = 4

    def body(i, _):
        pltpu.make_async_copy(ones_v, acc_sh.at[idx_v.at[i]], sem).start(
            add=True)

        @pl.when(i >= WIN)
        def _():
            pltpu.make_async_copy(ones_v, acc_sh.at[idx_v.at[i]], sem).wait()

        return 0

    lax.fori_loop(0, NCHUNK, body, 0, unroll=False)
    for _ in range(WIN):
        pltpu.make_async_copy(ones_v, acc_sh.at[idx_v.at[0]], sem).wait()
    plsc.subcore_barrier()

    for j in range(RPT // RSTAGE):
        rs = pl.ds(s * RPT + j * RSTAGE, RSTAGE)
        pltpu.make_async_copy(acc_sh.at[rs], out_hbm.at[c, rs], sem).start()
    for j in range(RPT // RSTAGE):
        rs = pl.ds(s * RPT + j * RSTAGE, RSTAGE)
        pltpu.make_async_copy(acc_sh.at[rs], out_hbm.at[c, rs], sem).wait()


@functools.partial(
    pl.kernel,
    out_type=jax.ShapeDtypeStruct((NC, NPAD, D), jnp.float32),
    mesh=_MESH,
    scratch_types=[
        pltpu.VMEM((2, 2, CB, K), jnp.int32),
        pltpu.VMEM((K, D), jnp.float32),
        pltpu.VMEM((K, D), jnp.float32),
        pltpu.VMEM((K, D), jnp.float32),
        pltpu.VMEM_SHARED((NPAD, D), jnp.float32),
        pltpu.SemaphoreType.DMA,
        pltpu.SemaphoreType.DMA,
        pltpu.SemaphoreType.DMA,
        pltpu.SemaphoreType.DMA,
    ],
)
def _sc_scatter(g_hbm, idx_hbm, out_hbm,
                idx_v, rows0, rows1, rows2, acc_sh, gsem0, gsem1, ssem, isem):
    c = lax.axis_index("c")
    s = lax.axis_index("s")
    wid = c * NS + s
    bufs = (rows0, rows1, rows2)
    gsems = (gsem0, gsem1)

    _zero_f32(rows0, K, D)
    for j in range(RPT // K):
        rs = pl.ds(s * RPT + j * K, K)
        pltpu.make_async_copy(rows0, acc_sh.at[rs], gsem0).start()
    for j in range(RPT // K):
        rs = pl.ds(s * RPT + j * K, K)
        pltpu.make_async_copy(rows0, acc_sh.at[rs], gsem0).wait()
    plsc.subcore_barrier()

    def g_start(bb, i, buf, gs):
        pltpu.make_async_copy(g_hbm.at[idx_v.at[bb, 0, i]], buf, gs).start()

    def g_wait(bb, i, buf, gs):
        pltpu.make_async_copy(g_hbm.at[idx_v.at[bb, 0, i]], buf, gs).wait()

    def s_start(bb, i, buf):
        pltpu.make_async_copy(buf, acc_sh.at[idx_v.at[bb, 1, i]],
                              ssem).start(add=True)

    def s_wait(bb, i, buf):
        pltpu.make_async_copy(buf, acc_sh.at[idx_v.at[bb, 1, i]], ssem).wait()

    pltpu.sync_copy(idx_hbm.at[wid, 0], idx_v.at[0])
    for blk in range(NBLK):
        bb = blk % 2

        g_start(bb, 0, rows0, gsem0)
        g_start(bb, 1, rows1, gsem1)
        if blk + 1 < NBLK:
            pltpu.make_async_copy(idx_hbm.at[wid, blk + 1],
                                  idx_v.at[1 - bb], isem).start()

        def six(q, _):
            base = 6 * q
            for u in range(6):
                i = base + u
                buf = bufs[u % 3]
                nbuf = bufs[(u + 2) % 3]
                gs = gsems[u % 2]
                g_wait(bb, i, buf, gs)
                if u == 0:
                    @pl.when(q > 0)
                    def _():
                        s_wait(bb, i - 1, bufs[2])
                else:
                    s_wait(bb, i - 1, bufs[(u - 1) % 3])

                @pl.when(i + 2 < CB)
                def _():
                    g_start(bb, i + 2, nbuf, gs)

                s_start(bb, i, buf)
            return 0

        lax.fori_loop(0, (CB - 1) // 6, six, 0, unroll=False)
        g_wait(bb, CB - 1, rows0, gsem0)
        s_wait(bb, CB - 2, rows2)
        s_start(bb, CB - 1, rows0)
        s_wait(bb, CB - 1, rows0)
        if blk + 1 < NBLK:
            pltpu.make_async_copy(idx_hbm.at[wid, blk + 1],
                                  idx_v.at[1 - bb], isem).wait()
    plsc.subcore_barrier()

    for j in range(RPT // RSTAGE):
        rs = pl.ds(s * RPT + j * RSTAGE, RSTAGE)
        pltpu.make_async_copy(acc_sh.at[rs], out_hbm.at[c, rs], gsem0).start()
    for j in range(RPT // RSTAGE):
        rs = pl.ds(s * RPT + j * RSTAGE, RSTAGE)
        pltpu.make_async_copy(acc_sh.at[rs], out_hbm.at[c, rs], gsem0).wait()


R = 1000


def _dinv_block(degp_ref):
    deg = degp_ref[0, :, 0:1] + degp_ref[1, :, 0:1] + 1.0
    return lax.rsqrt(deg)


def _tc1_body(x_ref, w_ref, degp_ref, g_ref):
    dinv = _dinv_block(degp_ref)
    g_ref[...] = jnp.dot(x_ref[...], w_ref[...],
                         preferred_element_type=jnp.float32) * dinv


def _tc2_body(ap_ref, g_ref, degp_ref, b_ref, w_ref, g2_ref):
    dinv = _dinv_block(degp_ref)
    pre = dinv * (ap_ref[0] + ap_ref[1] + g_ref[...]) + b_ref[...]
    x2 = jnp.maximum(pre, 0.0)
    g2_ref[...] = jnp.dot(x2, w_ref[...],
                          preferred_element_type=jnp.float32) * dinv


def _tc3_body(ap_ref, g_ref, degp_ref, b_ref, out_ref):
    dinv = _dinv_block(degp_ref)
    out_ref[...] = dinv * (ap_ref[0] + ap_ref[1] + g_ref[...]) + b_ref[...]


def _row_spec(r):
    return pl.BlockSpec((r, D), lambda i: (i, 0))


_pair_spec = pl.BlockSpec((NC, R, D), lambda i: (0, i, 0))
_degp_spec = _pair_spec
_full_w = pl.BlockSpec((D, D), lambda i: (0, 0))
_full_b = pl.BlockSpec((1, D), lambda i: (0, 0))
_out_rd = jax.ShapeDtypeStruct((N, D), jnp.float32)

_tc1 = pl.pallas_call(
    _tc1_body,
    grid=(N // R,),
    in_specs=[_row_spec(R), _full_w, _degp_spec],
    out_specs=_row_spec(R),
    out_shape=_out_rd,
)

_tc2 = pl.pallas_call(
    _tc2_body,
    grid=(N // R,),
    in_specs=[_pair_spec, _row_spec(R), _degp_spec, _full_b, _full_w],
    out_specs=_row_spec(R),
    out_shape=_out_rd,
)

_tc3 = pl.pallas_call(
    _tc3_body,
    grid=(N // R,),
    in_specs=[_pair_spec, _row_spec(R), _degp_spec, _full_b],
    out_specs=_row_spec(R),
    out_shape=_out_rd,
)


def kernel(basic_block, edge_index, W1, b1, W2, b2):
    ei4 = edge_index.reshape(2, NW, NBLK, CB, K)
    idx5 = ei4.transpose(1, 2, 0, 3, 4)
    dst3 = edge_index[1].reshape(NW, NCHUNK, K)
    b1r = b1.reshape(1, D)
    b2r = b2.reshape(1, D)

    degp = _sc_count(dst3)
    g1 = _tc1(basic_block, W1, degp)
    a1p = _sc_scatter(g1, idx5)
    g2 = _tc2(a1p, g1, degp, b1r, W2)
    a2p = _sc_scatter(g2, idx5)
    return _tc3(a2p, g2, degp, b2r)
